# baseline (speedup 1.0000x reference)
import functools

import jax
import jax.numpy as jnp
from jax import lax
from jax.experimental import pallas as pl
from jax.experimental.pallas import tpu as pltpu
from jax.experimental.pallas import tpu_sc as plsc

N_NODES = 10000
N_EDGES = 320000
D = 128

NC = 2
NS = 16
NW = NC * NS

CHUNK = 128
N_PAD = 10240
N_ACC = 10496
DUMMY = N_PAD
E_PER_W = None

LN_EPS = 1e-5


def _fill(ref, rows, width, value):
  nvec = width // 16
  val = jnp.full((16,), value, jnp.float32)

  def body(t, _):
    i = t // nvec
    j = t % nvec
    ref[i, pl.ds(j * 16, 16)] = val
    return _

  lax.fori_loop(0, rows * nvec, body, None)


def _zero_shared(acc, zbuf, s, rows_per_w, width):
  base = s * rows_per_w
  nfull = rows_per_w // 128
  rem = rows_per_w - nfull * 128
  for k in range(nfull):
    pltpu.sync_copy(zbuf, acc.at[pl.ds(base + k * 128, 128)])
  if rem:
    pltpu.sync_copy(zbuf.at[pl.ds(0, rem)], acc.at[pl.ds(base + nfull * 128, rem)])


K = 2
DEG_WIN = 8


def _fill_row(ref, r, width, val):
  for j in range(width // 16):
    ref[r, pl.ds(j * 16, 16)] = val


def _make_segsum(e_pad, with_deg=False):
  e_per_w = e_pad // NW
  n_chunks = e_per_w // CHUNK
  mesh = plsc.VectorSubcoreMesh(core_axis_name="c", subcore_axis_name="s")

  out_s_t = jax.ShapeDtypeStruct((NC, N_PAD, D), jnp.float32)
  out_type = (out_s_t, out_s_t) if with_deg else out_s_t
  scratch = [
      pltpu.VMEM((CHUNK,), jnp.int32),
      pltpu.VMEM((CHUNK,), jnp.int32),
      pltpu.VMEM((CHUNK, D), jnp.float32),
      pltpu.VMEM((CHUNK, D), jnp.float32),
      pltpu.VMEM_SHARED((N_ACC, D), jnp.float32),
      pltpu.SemaphoreType.DMA,
  ]

  def body(h, src, dst, *rest):
    if with_deg:
      out_s, out_d, sidx, didx, rows, ones, acc, sem = rest
    else:
      out_s, sidx, didx, rows, ones, acc, sem = rest
    c = lax.axis_index("c")
    s = lax.axis_index("s")
    wid = c * NS + s
    ob = s * (N_PAD // NS)
    base0 = wid * e_per_w

    _fill(rows, CHUNK, D, 0.0)
    _zero_shared(acc, rows, s, N_ACC // NS, D)
    plsc.subcore_barrier()

    if with_deg:
      _fill(ones, CHUNK, D, 1.0)

      def dstep(i, _):
        b = base0 + i * CHUNK
        pltpu.sync_copy(dst.at[pl.ds(b, CHUNK)], didx)
        pltpu.sync_copy(ones, acc.at[didx], add=True)
        return _

      lax.fori_loop(0, n_chunks, dstep, None)
      plsc.subcore_barrier()
      pltpu.sync_copy(acc.at[pl.ds(ob, N_PAD // NS)],
                      out_d.at[c, pl.ds(ob, N_PAD // NS)])
      plsc.subcore_barrier()
      _zero_shared(acc, rows, s, N_ACC // NS, D)
      plsc.subcore_barrier()

    def step(i, _):
      b = base0 + i * CHUNK
      pltpu.sync_copy(src.at[pl.ds(b, CHUNK)], sidx)
      pltpu.sync_copy(dst.at[pl.ds(b, CHUNK)], didx)
      pltpu.async_copy(h.at[sidx], rows, sem).wait()
      pltpu.sync_copy(rows, acc.at[didx], add=True)
      return _

    lax.fori_loop(0, n_chunks, step, None)
    plsc.subcore_barrier()

    pltpu.sync_copy(acc.at[pl.ds(ob, N_PAD // NS)],
                    out_s.at[c, pl.ds(ob, N_PAD // NS)])

  return pl.kernel(body, out_type=out_type, mesh=mesh, scratch_types=scratch)


def _make_deg(e_pad):
  e_per_w = e_pad // NW
  n_chunks = e_per_w // CHUNK
  mesh = plsc.VectorSubcoreMesh(core_axis_name="c", subcore_axis_name="s")

  out_type = jax.ShapeDtypeStruct((NC, N_PAD, D), jnp.float32)
  scratch = [
      pltpu.VMEM((CHUNK,), jnp.int32),
      pltpu.VMEM((CHUNK, D), jnp.float32),
      pltpu.VMEM_SHARED((N_ACC, D), jnp.float32),
      pltpu.SemaphoreType.DMA,
  ]

  def body(dst, out_d, didx, rows, acc, sem):
    c = lax.axis_index("c")
    s = lax.axis_index("s")
    wid = c * NS + s
    base0 = wid * e_per_w

    _fill(rows, CHUNK, D, 0.0)
    _zero_shared(acc, rows, s, N_ACC // NS, D)
    _fill(rows, CHUNK, D, 1.0)
    plsc.subcore_barrier()

    def step(i, _):
      b = base0 + i * CHUNK
      pltpu.sync_copy(dst.at[pl.ds(b, CHUNK)], didx)
      pltpu.sync_copy(rows, acc.at[didx], add=True)
      return _

    lax.fori_loop(0, n_chunks, step, None)
    plsc.subcore_barrier()

    ob = s * (N_PAD // NS)
    pltpu.sync_copy(acc.at[pl.ds(ob, N_PAD // NS)],
                    out_d.at[c, pl.ds(ob, N_PAD // NS)])

  return pl.kernel(body, out_type=out_type, mesh=mesh, scratch_types=scratch)


def _make_egather(e_pad):
  e_per_w = e_pad // NW
  n_chunks = e_per_w // CHUNK
  mesh = plsc.VectorSubcoreMesh(core_axis_name="c", subcore_axis_name="s")

  out_type = (jax.ShapeDtypeStruct((e_pad, D), jnp.float32),
              jax.ShapeDtypeStruct((e_pad, D), jnp.float32))
  scratch = [
      pltpu.VMEM((CHUNK,), jnp.int32),
      pltpu.VMEM((CHUNK,), jnp.int32),
      pltpu.VMEM((CHUNK, D), jnp.float32),
      pltpu.VMEM((CHUNK, D), jnp.float32),
      pltpu.SemaphoreType.DMA,
      pltpu.SemaphoreType.DMA,
  ]

  def body(p, q, src, dst, out_p, out_q, sidx, didx, arow, brow, sem_a, sem_b):
    c = lax.axis_index("c")
    s = lax.axis_index("s")
    wid = c * NS + s
    base0 = wid * e_per_w

    def step(i, _):
      b = base0 + i * CHUNK
      pltpu.sync_copy(src.at[pl.ds(b, CHUNK)], sidx)
      pltpu.sync_copy(dst.at[pl.ds(b, CHUNK)], didx)
      cp_a = pltpu.async_copy(p.at[sidx], arow, sem_a)
      cp_b = pltpu.async_copy(q.at[didx], brow, sem_b)
      cp_a.wait()
      pltpu.sync_copy(arow, out_p.at[pl.ds(b, CHUNK)])
      cp_b.wait()
      pltpu.sync_copy(brow, out_q.at[pl.ds(b, CHUNK)])
      return _

    lax.fori_loop(0, n_chunks, step, None)

  return pl.kernel(body, out_type=out_type, mesh=mesh, scratch_types=scratch)


def _layer_norm_rows(z):
  mu = jnp.mean(z, axis=-1, keepdims=True)
  zc = z - mu
  var = jnp.mean(zc * zc, axis=-1, keepdims=True)
  return zc * lax.rsqrt(var + LN_EPS)


def _tc_layer(h, s0, s1, d0, d1, w_s, w_n, rflag):
  blk = 1024
  grid = (N_PAD // blk,)

  def body(h_r, s0_r, s1_r, d0_r, d1_r, wn_r, ws_r, rf_r, o_r):
    deg = jnp.maximum(d0_r[...][:, :1] + d1_r[...][:, :1], 1.0)
    ssum = s0_r[...] + s1_r[...]
    z = (jnp.dot(ssum, wn_r[...], preferred_element_type=jnp.float32) / deg
         + jnp.dot(h_r[...], ws_r[...], preferred_element_type=jnp.float32))
    z = _layer_norm_rows(z)
    z = jnp.where(rf_r[0, 0] > 0.0, jnp.maximum(z, 0.0), z)
    o_r[...] = z + h_r[...]

  node = pl.BlockSpec((blk, D), lambda i: (i, 0))
  node16 = pl.BlockSpec((blk, 16), lambda i: (i, 0))
  full = pl.BlockSpec((D, D), lambda i: (0, 0))
  one = pl.BlockSpec((1, 1), lambda i: (0, 0))
  return pl.pallas_call(
      body,
      grid=grid,
      in_specs=[node, node, node, node16, node16, full, full, one],
      out_specs=node,
      out_shape=jax.ShapeDtypeStruct((N_PAD, D), jnp.float32),
  )(h, s0, s1, d0, d1, w_n, w_s, rflag)


def _tc_dec(h, dec_w1, dec_b1, dec_w2, dec_b2, ed_w1a, ed_w1b, ed_b1):
  blk = 1024
  grid = (N_PAD // blk,)

  def body(h_r, dw1_r, db1_r, dw2_r, db2_r, ea_r, eb_r, eb1_r,
           rec_r, p_r, q_r):
    z = h_r[...]
    t = jnp.dot(z, dw1_r[...], preferred_element_type=jnp.float32) + db1_r[...]
    t = jnp.maximum(_layer_norm_rows(t), 0.0)
    rec_r[...] = jnp.dot(t, dw2_r[...], preferred_element_type=jnp.float32) + db2_r[...]
    p_r[...] = jnp.dot(z, ea_r[...], preferred_element_type=jnp.float32) + eb1_r[...]
    q_r[...] = jnp.dot(z, eb_r[...], preferred_element_type=jnp.float32)

  node = pl.BlockSpec((blk, D), lambda i: (i, 0))
  full = pl.BlockSpec((D, D), lambda i: (0, 0))
  brow = pl.BlockSpec((1, D), lambda i: (0, 0))
  out = jax.ShapeDtypeStruct((N_PAD, D), jnp.float32)
  return pl.pallas_call(
      body,
      grid=grid,
      in_specs=[node, full, brow, full, brow, full, full, brow],
      out_specs=(node, node, node),
      out_shape=(out, out, out),
  )(h, dec_w1, dec_b1, dec_w2, dec_b2, ed_w1a, ed_w1b, ed_b1)


def _tc_edge(e_pad, up, uq, ed_w2, ed_b2):
  blk = 2048
  grid = (e_pad // blk,)
  nrel = ed_w2.shape[1]

  def body(up_r, uq_r, w2_r, b2_r, o_r):
    u = up_r[...] + uq_r[...]
    u = jnp.maximum(_layer_norm_rows(u), 0.0)
    o_r[...] = jnp.dot(u, w2_r[...], preferred_element_type=jnp.float32) + b2_r[...]

  edge = pl.BlockSpec((blk, D), lambda i: (i, 0))
  return pl.pallas_call(
      body,
      grid=grid,
      in_specs=[edge, edge,
                pl.BlockSpec((D, nrel), lambda i: (0, 0)),
                pl.BlockSpec((1, nrel), lambda i: (0, 0))],
      out_specs=pl.BlockSpec((blk, nrel), lambda i: (i, 0)),
      out_shape=jax.ShapeDtypeStruct((e_pad, nrel), jnp.float32),
  )(up, uq, ed_w2, ed_b2)


def kernel(x, edge_index, W_self0, W_nbr0, W_self1, W_nbr1, W_self2, W_nbr2,
           dec_w1, dec_b1, dec_w2, dec_b2, ed_w1, ed_b1, ed_w2, ed_b2):
  n, d = x.shape
  e = edge_index.shape[1]
  quantum = NW * CHUNK * K * DEG_WIN
  e_pad = ((e + quantum - 1) // quantum) * quantum

  src = jnp.pad(edge_index[0], (0, e_pad - e))
  dst = jnp.pad(edge_index[1], (0, e_pad - e),
                constant_values=DUMMY)
  src2 = src.reshape(e_pad // CHUNK, CHUNK)
  dst2 = dst.reshape(e_pad // CHUNK, CHUNK)
  xp = jnp.pad(x, ((0, N_PAD - n), (0, 0)))

  segsum = _make_segsum(e_pad)
  deg_k = _make_deg(e_pad)
  egather = _make_egather(e_pad)

  d_full = deg_k(dst)
  d_parts = d_full[:, :, :16]
  s_parts = segsum(xp, src, dst)
  rflags = jnp.ones((3, 1, 1), jnp.float32).at[2].set(0.0)
  h1 = _tc_layer(xp, s_parts[0], s_parts[1], d_parts[0], d_parts[1],
                 W_self0, W_nbr0, rflags[0])
  s_parts1 = segsum(h1, src, dst)
  h2 = _tc_layer(h1, s_parts1[0], s_parts1[1], d_parts[0], d_parts[1],
                 W_self1, W_nbr1, rflags[1])
  s_parts2 = segsum(h2, src, dst)
  h3 = _tc_layer(h2, s_parts2[0], s_parts2[1], d_parts[0], d_parts[1],
                 W_self2, W_nbr2, rflags[2])
  recon_p, p_nodes, q_nodes = _tc_dec(
      h3, dec_w1, dec_b1.reshape(1, -1), dec_w2, dec_b2.reshape(1, -1),
      ed_w1[:d], ed_w1[d:], ed_b1.reshape(1, -1))
  up, uq = egather(p_nodes, q_nodes, src, dst)
  logits_p = _tc_edge(e_pad, up, uq, ed_w2, ed_b2.reshape(1, -1))
  return (recon_p[:n], logits_p[:e])


# speedup vs baseline: 1.3414x; 1.3414x over previous
import functools

import jax
import jax.numpy as jnp
from jax import lax
from jax.experimental import pallas as pl
from jax.experimental.pallas import tpu as pltpu
from jax.experimental.pallas import tpu_sc as plsc

N_NODES = 10000
N_EDGES = 320000
D = 128

NC = 2
NS = 16
NW = NC * NS

CHUNK = 128
N_PAD = 10240
N_ACC = 10496
DUMMY = N_PAD
E_PER_W = None

LN_EPS = 1e-5


def _fill(ref, rows, width, value):
  nvec = width // 16
  val = jnp.full((16,), value, jnp.float32)

  def body(t, _):
    i = t // nvec
    j = t % nvec
    ref[i, pl.ds(j * 16, 16)] = val
    return _

  lax.fori_loop(0, rows * nvec, body, None)


def _zero_shared(acc, zbuf, s, rows_per_w, width):
  base = s * rows_per_w
  nfull = rows_per_w // 128
  rem = rows_per_w - nfull * 128
  for k in range(nfull):
    pltpu.sync_copy(zbuf, acc.at[pl.ds(base + k * 128, 128)])
  if rem:
    pltpu.sync_copy(zbuf.at[pl.ds(0, rem)], acc.at[pl.ds(base + nfull * 128, rem)])


K = 2
DEG_WIN = 8


def _fill_row(ref, r, width, val):
  for j in range(width // 16):
    ref[r, pl.ds(j * 16, 16)] = val


def _make_segsum(e_pad, with_deg=False):
  e_per_w = e_pad // NW
  n_chunks = e_per_w // CHUNK
  mesh = plsc.VectorSubcoreMesh(core_axis_name="c", subcore_axis_name="s")

  out_s_t = jax.ShapeDtypeStruct((NC, N_PAD, D), jnp.float32)
  out_type = (out_s_t, out_s_t) if with_deg else out_s_t
  scratch = [
      pltpu.VMEM((CHUNK,), jnp.int32),
      pltpu.VMEM((CHUNK,), jnp.int32),
      pltpu.VMEM((CHUNK, D), jnp.float32),
      pltpu.VMEM((CHUNK, D), jnp.float32),
      pltpu.VMEM_SHARED((N_ACC, D), jnp.float32),
      pltpu.SemaphoreType.DMA,
  ]

  def body(h, src, dst, *rest):
    if with_deg:
      out_s, out_d, sidx, didx, rows, ones, acc, sem = rest
    else:
      out_s, sidx, didx, rows, ones, acc, sem = rest
    c = lax.axis_index("c")
    s = lax.axis_index("s")
    wid = c * NS + s
    ob = s * (N_PAD // NS)
    base0 = wid * e_per_w

    _fill(rows, CHUNK, D, 0.0)
    _zero_shared(acc, rows, s, N_ACC // NS, D)
    plsc.subcore_barrier()

    if with_deg:
      _fill(ones, CHUNK, D, 1.0)

      def dstep(i, _):
        b = base0 + i * CHUNK
        pltpu.sync_copy(dst.at[pl.ds(b, CHUNK)], didx)
        pltpu.sync_copy(ones, acc.at[didx], add=True)
        return _

      lax.fori_loop(0, n_chunks, dstep, None)
      plsc.subcore_barrier()
      pltpu.sync_copy(acc.at[pl.ds(ob, N_PAD // NS)],
                      out_d.at[c, pl.ds(ob, N_PAD // NS)])
      plsc.subcore_barrier()
      _zero_shared(acc, rows, s, N_ACC // NS, D)
      plsc.subcore_barrier()

    def step(i, _):
      b = base0 + i * CHUNK
      pltpu.sync_copy(src.at[pl.ds(b, CHUNK)], sidx)
      pltpu.sync_copy(dst.at[pl.ds(b, CHUNK)], didx)
      pltpu.async_copy(h.at[sidx], rows, sem).wait()
      pltpu.sync_copy(rows, acc.at[didx], add=True)
      return _

    lax.fori_loop(0, n_chunks, step, None)
    plsc.subcore_barrier()

    pltpu.sync_copy(acc.at[pl.ds(ob, N_PAD // NS)],
                    out_s.at[c, pl.ds(ob, N_PAD // NS)])

  return pl.kernel(body, out_type=out_type, mesh=mesh, scratch_types=scratch)


def _make_deg(e_pad):
  e_per_w = e_pad // NW
  n_chunks = e_per_w // CHUNK
  mesh = plsc.VectorSubcoreMesh(core_axis_name="c", subcore_axis_name="s")

  out_type = jax.ShapeDtypeStruct((NC, N_PAD, D), jnp.float32)
  scratch = [
      pltpu.VMEM((CHUNK,), jnp.int32),
      pltpu.VMEM((CHUNK, D), jnp.float32),
      pltpu.VMEM_SHARED((N_ACC, D), jnp.float32),
      pltpu.SemaphoreType.DMA,
  ]

  def body(dst, out_d, didx, rows, acc, sem):
    c = lax.axis_index("c")
    s = lax.axis_index("s")
    wid = c * NS + s
    base0 = wid * e_per_w

    _fill(rows, CHUNK, D, 0.0)
    _zero_shared(acc, rows, s, N_ACC // NS, D)
    _fill(rows, CHUNK, D, 1.0)
    plsc.subcore_barrier()

    def step(i, _):
      b = base0 + i * CHUNK
      pltpu.sync_copy(dst.at[pl.ds(b, CHUNK)], didx)
      pltpu.sync_copy(rows, acc.at[didx], add=True)
      return _

    lax.fori_loop(0, n_chunks, step, None)
    plsc.subcore_barrier()

    ob = s * (N_PAD // NS)
    pltpu.sync_copy(acc.at[pl.ds(ob, N_PAD // NS)],
                    out_d.at[c, pl.ds(ob, N_PAD // NS)])

  return pl.kernel(body, out_type=out_type, mesh=mesh, scratch_types=scratch)


def _make_egather(e_pad):
  e_per_w = e_pad // NW
  n_chunks = e_per_w // CHUNK
  mesh = plsc.VectorSubcoreMesh(core_axis_name="c", subcore_axis_name="s")

  out_type = (jax.ShapeDtypeStruct((e_pad, D), jnp.float32),
              jax.ShapeDtypeStruct((e_pad, D), jnp.float32))
  scratch = [
      pltpu.VMEM((CHUNK,), jnp.int32),
      pltpu.VMEM((CHUNK,), jnp.int32),
      pltpu.VMEM((CHUNK, D), jnp.float32),
      pltpu.VMEM((CHUNK, D), jnp.float32),
      pltpu.SemaphoreType.DMA,
      pltpu.SemaphoreType.DMA,
  ]

  def body(p, q, src, dst, out_p, out_q, sidx, didx, arow, brow, sem_a, sem_b):
    c = lax.axis_index("c")
    s = lax.axis_index("s")
    wid = c * NS + s
    base0 = wid * e_per_w

    def step(i, _):
      b = base0 + i * CHUNK
      pltpu.sync_copy(src.at[pl.ds(b, CHUNK)], sidx)
      pltpu.sync_copy(dst.at[pl.ds(b, CHUNK)], didx)
      cp_a = pltpu.async_copy(p.at[sidx], arow, sem_a)
      cp_b = pltpu.async_copy(q.at[didx], brow, sem_b)
      cp_a.wait()
      pltpu.sync_copy(arow, out_p.at[pl.ds(b, CHUNK)])
      cp_b.wait()
      pltpu.sync_copy(brow, out_q.at[pl.ds(b, CHUNK)])
      return _

    lax.fori_loop(0, n_chunks, step, None)

  return pl.kernel(body, out_type=out_type, mesh=mesh, scratch_types=scratch)


def _layer_norm_rows(z):
  mu = jnp.mean(z, axis=-1, keepdims=True)
  zc = z - mu
  var = jnp.mean(zc * zc, axis=-1, keepdims=True)
  return zc * lax.rsqrt(var + LN_EPS)


def _tc_layer(h, s0, s1, d0, d1, w_s, w_n, rflag):
  blk = 1024
  grid = (N_PAD // blk,)

  def body(h_r, s0_r, s1_r, d0_r, d1_r, wn_r, ws_r, rf_r, o_r):
    deg = jnp.maximum(d0_r[...][:, :1] + d1_r[...][:, :1], 1.0)
    ssum = s0_r[...] + s1_r[...]
    z = (jnp.dot(ssum, wn_r[...], preferred_element_type=jnp.float32) / deg
         + jnp.dot(h_r[...], ws_r[...], preferred_element_type=jnp.float32))
    z = _layer_norm_rows(z)
    z = jnp.where(rf_r[0, 0] > 0.0, jnp.maximum(z, 0.0), z)
    o_r[...] = z + h_r[...]

  node = pl.BlockSpec((blk, D), lambda i: (i, 0))
  node16 = pl.BlockSpec((blk, 16), lambda i: (i, 0))
  full = pl.BlockSpec((D, D), lambda i: (0, 0))
  one = pl.BlockSpec((1, 1), lambda i: (0, 0))
  return pl.pallas_call(
      body,
      grid=grid,
      in_specs=[node, node, node, node16, node16, full, full, one],
      out_specs=node,
      out_shape=jax.ShapeDtypeStruct((N_PAD, D), jnp.float32),
  )(h, s0, s1, d0, d1, w_n, w_s, rflag)


def _tc_dec(h, dec_w1, dec_b1, dec_w2, dec_b2, ed_w1a, ed_w1b, ed_b1):
  blk = 1024
  grid = (N_PAD // blk,)

  def body(h_r, dw1_r, db1_r, dw2_r, db2_r, ea_r, eb_r, eb1_r,
           rec_r, p_r, q_r):
    z = h_r[...]
    t = jnp.dot(z, dw1_r[...], preferred_element_type=jnp.float32) + db1_r[...]
    t = jnp.maximum(_layer_norm_rows(t), 0.0)
    rec_r[...] = jnp.dot(t, dw2_r[...], preferred_element_type=jnp.float32) + db2_r[...]
    p_r[...] = jnp.dot(z, ea_r[...], preferred_element_type=jnp.float32) + eb1_r[...]
    q_r[...] = jnp.dot(z, eb_r[...], preferred_element_type=jnp.float32)

  node = pl.BlockSpec((blk, D), lambda i: (i, 0))
  full = pl.BlockSpec((D, D), lambda i: (0, 0))
  brow = pl.BlockSpec((1, D), lambda i: (0, 0))
  out = jax.ShapeDtypeStruct((N_PAD, D), jnp.float32)
  return pl.pallas_call(
      body,
      grid=grid,
      in_specs=[node, full, brow, full, brow, full, full, brow],
      out_specs=(node, node, node),
      out_shape=(out, out, out),
  )(h, dec_w1, dec_b1, dec_w2, dec_b2, ed_w1a, ed_w1b, ed_b1)


def _tc_edge(e_pad, up, uq, ed_w2, ed_b2):
  blk = 2048
  grid = (e_pad // blk,)
  nrel = ed_w2.shape[1]

  def body(up_r, uq_r, w2_r, b2_r, o_r):
    u = up_r[...] + uq_r[...]
    u = jnp.maximum(_layer_norm_rows(u), 0.0)
    o_r[...] = jnp.dot(u, w2_r[...], preferred_element_type=jnp.float32) + b2_r[...]

  edge = pl.BlockSpec((blk, D), lambda i: (i, 0))
  return pl.pallas_call(
      body,
      grid=grid,
      in_specs=[edge, edge,
                pl.BlockSpec((D, nrel), lambda i: (0, 0)),
                pl.BlockSpec((1, nrel), lambda i: (0, 0))],
      out_specs=pl.BlockSpec((blk, nrel), lambda i: (i, 0)),
      out_shape=jax.ShapeDtypeStruct((e_pad, nrel), jnp.float32),
  )(up, uq, ed_w2, ed_b2)


def kernel(x, edge_index, W_self0, W_nbr0, W_self1, W_nbr1, W_self2, W_nbr2,
           dec_w1, dec_b1, dec_w2, dec_b2, ed_w1, ed_b1, ed_w2, ed_b2):
  n, d = x.shape
  e = edge_index.shape[1]
  quantum = NW * CHUNK
  e_pad = ((e + quantum - 1) // quantum) * quantum

  src = jnp.pad(edge_index[0], (0, e_pad - e))
  dst = jnp.pad(edge_index[1], (0, e_pad - e),
                constant_values=DUMMY)
  src2 = src.reshape(e_pad // CHUNK, CHUNK)
  dst2 = dst.reshape(e_pad // CHUNK, CHUNK)
  xp = jnp.pad(x, ((0, N_PAD - n), (0, 0)))

  segsum = _make_segsum(e_pad)
  deg_k = _make_deg(e_pad)
  egather = _make_egather(e_pad)

  d_full = deg_k(dst)
  d_parts = d_full[:, :, :16]
  s_parts = segsum(xp, src, dst)
  rflags = jnp.asarray([[[1.0]], [[1.0]], [[0.0]]], jnp.float32)
  h1 = _tc_layer(xp, s_parts[0], s_parts[1], d_parts[0], d_parts[1],
                 W_self0, W_nbr0, rflags[0])
  s_parts1 = segsum(h1, src, dst)
  h2 = _tc_layer(h1, s_parts1[0], s_parts1[1], d_parts[0], d_parts[1],
                 W_self1, W_nbr1, rflags[1])
  s_parts2 = segsum(h2, src, dst)
  h3 = _tc_layer(h2, s_parts2[0], s_parts2[1], d_parts[0], d_parts[1],
                 W_self2, W_nbr2, rflags[2])
  recon_p, p_nodes, q_nodes = _tc_dec(
      h3, dec_w1, dec_b1.reshape(1, -1), dec_w2, dec_b2.reshape(1, -1),
      ed_w1[:d], ed_w1[d:], ed_b1.reshape(1, -1))
  up, uq = egather(p_nodes, q_nodes, src, dst)
  logits_p = _tc_edge(e_pad, up, uq, ed_w2, ed_b2.reshape(1, -1))
  return (recon_p[:n], logits_p[:e])


# speedup vs baseline: 1.3422x; 1.0006x over previous
import jax
import jax.numpy as jnp
from jax import lax
from jax.experimental import pallas as pl
from jax.experimental.pallas import tpu as pltpu
from jax.experimental.pallas import tpu_sc as plsc

N_NODES = 10000
N_EDGES = 320000
D = 128

NC = 2
NS = 16
NW = NC * NS

CHUNK = 128
N_PAD = 10240
N_ACC = 10496
DUMMY = N_PAD

LN_EPS = 1e-5


def _fill(ref, rows, width, value):
  nvec = width // 16
  val = jnp.full((16,), value, jnp.float32)

  def body(t, _):
    i = t // nvec
    j = t % nvec
    ref[i, pl.ds(j * 16, 16)] = val
    return _

  lax.fori_loop(0, rows * nvec, body, None)


def _zero_shared(acc, zbuf, s, rows_per_w, width):
  base = s * rows_per_w
  nfull = rows_per_w // 128
  rem = rows_per_w - nfull * 128
  for k in range(nfull):
    pltpu.sync_copy(zbuf, acc.at[pl.ds(base + k * 128, 128)])
  if rem:
    pltpu.sync_copy(zbuf.at[pl.ds(0, rem)], acc.at[pl.ds(base + nfull * 128, rem)])


def _make_segsum(e_pad, with_deg=False):
  e_per_w = e_pad // NW
  n_chunks = e_per_w // CHUNK
  mesh = plsc.VectorSubcoreMesh(core_axis_name="c", subcore_axis_name="s")

  out_s_t = jax.ShapeDtypeStruct((NC, N_PAD, D), jnp.float32)
  out_type = (out_s_t, out_s_t) if with_deg else out_s_t
  scratch = [
      pltpu.VMEM((CHUNK,), jnp.int32),
      pltpu.VMEM((CHUNK,), jnp.int32),
      pltpu.VMEM((CHUNK, D), jnp.float32),
      pltpu.VMEM((CHUNK, D), jnp.float32),
      pltpu.VMEM_SHARED((N_ACC, D), jnp.float32),
      pltpu.SemaphoreType.DMA,
  ]

  def body(h, src, dst, *rest):
    if with_deg:
      out_s, out_d, sidx, didx, rows, ones, acc, sem = rest
    else:
      out_s, sidx, didx, rows, ones, acc, sem = rest
    c = lax.axis_index("c")
    s = lax.axis_index("s")
    wid = c * NS + s
    ob = s * (N_PAD // NS)
    base0 = wid * e_per_w

    _fill(rows, CHUNK, D, 0.0)
    _zero_shared(acc, rows, s, N_ACC // NS, D)
    plsc.subcore_barrier()

    if with_deg:
      _fill(ones, CHUNK, D, 1.0)

      def dstep(i, _):
        b = base0 + i * CHUNK
        pltpu.sync_copy(dst.at[pl.ds(b, CHUNK)], didx)
        pltpu.sync_copy(ones, acc.at[didx], add=True)
        return _

      lax.fori_loop(0, n_chunks, dstep, None)
      plsc.subcore_barrier()
      pltpu.sync_copy(acc.at[pl.ds(ob, N_PAD // NS)],
                      out_d.at[c, pl.ds(ob, N_PAD // NS)])
      plsc.subcore_barrier()
      _zero_shared(acc, rows, s, N_ACC // NS, D)
      plsc.subcore_barrier()

    def step(i, _):
      b = base0 + i * CHUNK
      pltpu.sync_copy(src.at[pl.ds(b, CHUNK)], sidx)
      pltpu.sync_copy(dst.at[pl.ds(b, CHUNK)], didx)
      pltpu.async_copy(h.at[sidx], rows, sem).wait()
      pltpu.sync_copy(rows, acc.at[didx], add=True)
      return _

    lax.fori_loop(0, n_chunks, step, None)
    plsc.subcore_barrier()

    pltpu.sync_copy(acc.at[pl.ds(ob, N_PAD // NS)],
                    out_s.at[c, pl.ds(ob, N_PAD // NS)])

  return pl.kernel(body, out_type=out_type, mesh=mesh, scratch_types=scratch)


def _make_deg(e_pad):
  e_per_w = e_pad // NW
  n_chunks = e_per_w // CHUNK
  mesh = plsc.VectorSubcoreMesh(core_axis_name="c", subcore_axis_name="s")

  out_type = jax.ShapeDtypeStruct((NC, N_PAD, D), jnp.float32)
  scratch = [
      pltpu.VMEM((CHUNK,), jnp.int32),
      pltpu.VMEM((CHUNK, D), jnp.float32),
      pltpu.VMEM_SHARED((N_ACC, D), jnp.float32),
      pltpu.SemaphoreType.DMA,
  ]

  def body(dst, out_d, didx, rows, acc, sem):
    c = lax.axis_index("c")
    s = lax.axis_index("s")
    wid = c * NS + s
    base0 = wid * e_per_w

    _fill(rows, CHUNK, D, 0.0)
    _zero_shared(acc, rows, s, N_ACC // NS, D)
    _fill(rows, CHUNK, D, 1.0)
    plsc.subcore_barrier()

    def step(i, _):
      b = base0 + i * CHUNK
      pltpu.sync_copy(dst.at[pl.ds(b, CHUNK)], didx)
      pltpu.sync_copy(rows, acc.at[didx], add=True)
      return _

    lax.fori_loop(0, n_chunks, step, None)
    plsc.subcore_barrier()

    ob = s * (N_PAD // NS)
    pltpu.sync_copy(acc.at[pl.ds(ob, N_PAD // NS)],
                    out_d.at[c, pl.ds(ob, N_PAD // NS)])

  return pl.kernel(body, out_type=out_type, mesh=mesh, scratch_types=scratch)


def _make_egather(e_pad):
  e_per_w = e_pad // NW
  n_chunks = e_per_w // CHUNK
  mesh = plsc.VectorSubcoreMesh(core_axis_name="c", subcore_axis_name="s")

  out_type = (jax.ShapeDtypeStruct((e_pad, D), jnp.float32),
              jax.ShapeDtypeStruct((e_pad, D), jnp.float32))
  scratch = [
      pltpu.VMEM((CHUNK,), jnp.int32),
      pltpu.VMEM((CHUNK,), jnp.int32),
      pltpu.VMEM((CHUNK, D), jnp.float32),
      pltpu.VMEM((CHUNK, D), jnp.float32),
      pltpu.SemaphoreType.DMA,
      pltpu.SemaphoreType.DMA,
  ]

  def body(p, q, src, dst, out_p, out_q, sidx, didx, arow, brow, sem_a, sem_b):
    c = lax.axis_index("c")
    s = lax.axis_index("s")
    wid = c * NS + s
    base0 = wid * e_per_w

    def step(i, _):
      b = base0 + i * CHUNK
      pltpu.sync_copy(src.at[pl.ds(b, CHUNK)], sidx)
      pltpu.sync_copy(dst.at[pl.ds(b, CHUNK)], didx)
      cp_a = pltpu.async_copy(p.at[sidx], arow, sem_a)
      cp_b = pltpu.async_copy(q.at[didx], brow, sem_b)
      cp_a.wait()
      pltpu.sync_copy(arow, out_p.at[pl.ds(b, CHUNK)])
      cp_b.wait()
      pltpu.sync_copy(brow, out_q.at[pl.ds(b, CHUNK)])
      return _

    lax.fori_loop(0, n_chunks, step, None)

  return pl.kernel(body, out_type=out_type, mesh=mesh, scratch_types=scratch)


def _layer_norm_rows(z):
  mu = jnp.mean(z, axis=-1, keepdims=True)
  zc = z - mu
  var = jnp.mean(zc * zc, axis=-1, keepdims=True)
  return zc * lax.rsqrt(var + LN_EPS)


def _tc_layer(h, s0, s1, d0, d1, w_s, w_n, rflag):
  blk = 1024
  grid = (N_PAD // blk,)

  def body(h_r, s0_r, s1_r, d0_r, d1_r, wn_r, ws_r, rf_r, o_r):
    deg = jnp.maximum(d0_r[...][:, :1] + d1_r[...][:, :1], 1.0)
    ssum = s0_r[...] + s1_r[...]
    z = (jnp.dot(ssum, wn_r[...], preferred_element_type=jnp.float32) / deg
         + jnp.dot(h_r[...], ws_r[...], preferred_element_type=jnp.float32))
    z = _layer_norm_rows(z)
    z = jnp.where(rf_r[0, 0] > 0.0, jnp.maximum(z, 0.0), z)
    o_r[...] = z + h_r[...]

  node = pl.BlockSpec((blk, D), lambda i: (i, 0))
  node16 = pl.BlockSpec((blk, 16), lambda i: (i, 0))
  full = pl.BlockSpec((D, D), lambda i: (0, 0))
  one = pl.BlockSpec((1, 1), lambda i: (0, 0))
  return pl.pallas_call(
      body,
      grid=grid,
      in_specs=[node, node, node, node16, node16, full, full, one],
      out_specs=node,
      out_shape=jax.ShapeDtypeStruct((N_PAD, D), jnp.float32),
  )(h, s0, s1, d0, d1, w_n, w_s, rflag)


def _tc_dec(h, dec_w1, dec_b1, dec_w2, dec_b2, ed_w1a, ed_w1b, ed_b1):
  blk = 1024
  grid = (N_PAD // blk,)

  def body(h_r, dw1_r, db1_r, dw2_r, db2_r, ea_r, eb_r, eb1_r,
           rec_r, p_r, q_r):
    z = h_r[...]
    t = jnp.dot(z, dw1_r[...], preferred_element_type=jnp.float32) + db1_r[...]
    t = jnp.maximum(_layer_norm_rows(t), 0.0)
    rec_r[...] = jnp.dot(t, dw2_r[...], preferred_element_type=jnp.float32) + db2_r[...]
    p_r[...] = jnp.dot(z, ea_r[...], preferred_element_type=jnp.float32) + eb1_r[...]
    q_r[...] = jnp.dot(z, eb_r[...], preferred_element_type=jnp.float32)

  node = pl.BlockSpec((blk, D), lambda i: (i, 0))
  full = pl.BlockSpec((D, D), lambda i: (0, 0))
  brow = pl.BlockSpec((1, D), lambda i: (0, 0))
  out = jax.ShapeDtypeStruct((N_PAD, D), jnp.float32)
  return pl.pallas_call(
      body,
      grid=grid,
      in_specs=[node, full, brow, full, brow, full, full, brow],
      out_specs=(node, node, node),
      out_shape=(out, out, out),
  )(h, dec_w1, dec_b1, dec_w2, dec_b2, ed_w1a, ed_w1b, ed_b1)


def _tc_edge(e_pad, up, uq, ed_w2, ed_b2):
  blk = 2048
  grid = (e_pad // blk,)
  nrel = ed_w2.shape[1]

  def body(up_r, uq_r, w2_r, b2_r, o_r):
    u = up_r[...] + uq_r[...]
    u = jnp.maximum(_layer_norm_rows(u), 0.0)
    o_r[...] = jnp.dot(u, w2_r[...], preferred_element_type=jnp.float32) + b2_r[...]

  edge = pl.BlockSpec((blk, D), lambda i: (i, 0))
  return pl.pallas_call(
      body,
      grid=grid,
      in_specs=[edge, edge,
                pl.BlockSpec((D, nrel), lambda i: (0, 0)),
                pl.BlockSpec((1, nrel), lambda i: (0, 0))],
      out_specs=pl.BlockSpec((blk, nrel), lambda i: (i, 0)),
      out_shape=jax.ShapeDtypeStruct((e_pad, nrel), jnp.float32),
  )(up, uq, ed_w2, ed_b2)


def kernel(x, edge_index, W_self0, W_nbr0, W_self1, W_nbr1, W_self2, W_nbr2,
           dec_w1, dec_b1, dec_w2, dec_b2, ed_w1, ed_b1, ed_w2, ed_b2):
  n, d = x.shape
  e = edge_index.shape[1]
  quantum = NW * CHUNK
  e_pad = ((e + quantum - 1) // quantum) * quantum

  src = jnp.pad(edge_index[0], (0, e_pad - e))
  dst = jnp.pad(edge_index[1], (0, e_pad - e),
                constant_values=DUMMY)
  src2 = src.reshape(e_pad // CHUNK, CHUNK)
  dst2 = dst.reshape(e_pad // CHUNK, CHUNK)
  xp = jnp.pad(x, ((0, N_PAD - n), (0, 0)))

  segsum = _make_segsum(e_pad)
  deg_k = _make_deg(e_pad)
  egather = _make_egather(e_pad)

  d_full = deg_k(dst)
  d_parts = d_full[:, :, :16]
  s_parts = segsum(xp, src, dst)
  rflags = jnp.asarray([[[1.0]], [[1.0]], [[0.0]]], jnp.float32)
  h1 = _tc_layer(xp, s_parts[0], s_parts[1], d_parts[0], d_parts[1],
                 W_self0, W_nbr0, rflags[0])
  s_parts1 = segsum(h1, src, dst)
  h2 = _tc_layer(h1, s_parts1[0], s_parts1[1], d_parts[0], d_parts[1],
                 W_self1, W_nbr1, rflags[1])
  s_parts2 = segsum(h2, src, dst)
  h3 = _tc_layer(h2, s_parts2[0], s_parts2[1], d_parts[0], d_parts[1],
                 W_self2, W_nbr2, rflags[2])
  recon_p, p_nodes, q_nodes = _tc_dec(
      h3, dec_w1, dec_b1.reshape(1, -1), dec_w2, dec_b2.reshape(1, -1),
      ed_w1[:d], ed_w1[d:], ed_b1.reshape(1, -1))
  up, uq = egather(p_nodes, q_nodes, src, dst)
  logits_p = _tc_edge(e_pad, up, uq, ed_w2, ed_b2.reshape(1, -1))
  return (recon_p[:n], logits_p[:e])
